# baseline (speedup 1.0000x reference)
import functools

import jax
import jax.numpy as jnp
from jax import lax
from jax.experimental import pallas as pl
from jax.experimental.pallas import tpu as pltpu
from jax.experimental.pallas import tpu_sc as plsc

D_MODEL = 128
N_TABLE = 2048
NC, NS = 2, 16
NW = NC * NS
CHUNK = 64
NBUF = 8
LAG = 2
NSB = 2


def _sslot(b):
    return (b // 2) % NSB


def _make_gather(B):
    b_per_w = B // NW
    n_chunks = b_per_w // CHUNK
    assert n_chunks % NBUF == 0 and n_chunks > 2 * NBUF
    n_groups = n_chunks // NBUF
    mesh = plsc.VectorSubcoreMesh(core_axis_name="c", subcore_axis_name="s")

    @functools.partial(
        pl.kernel,
        out_type=jax.ShapeDtypeStruct((B, D_MODEL), jnp.float32),
        mesh=mesh,
        scratch_types=[
            pltpu.VMEM((b_per_w,), jnp.int32),
            pltpu.VMEM_SHARED((N_TABLE, D_MODEL), jnp.float32),
            pltpu.VMEM_SHARED((NS, NSB, CHUNK, D_MODEL), jnp.float32),
            *[pltpu.VMEM((CHUNK, D_MODEL), jnp.float32) for _ in range(NBUF)],
            *[pltpu.SemaphoreType.DMA for _ in range(NBUF)],
            *[pltpu.SemaphoreType.DMA for _ in range(NBUF // 2)],
            *[pltpu.SemaphoreType.DMA for _ in range(NBUF // 2)],
            *[pltpu.SemaphoreType.DMA for _ in range(NSB)],
        ],
    )
    def gather_kernel(idx_hbm, pe_hbm, out_hbm, idx_v, table_sh, sbuf_sh,
                      *bufs_and_sems):
        rows = bufs_and_sems[:NBUF]
        gsem = bufs_and_sems[NBUF:2 * NBUF]
        wsem = bufs_and_sems[2 * NBUF:2 * NBUF + NBUF // 2]
        s1sem = bufs_and_sems[2 * NBUF + NBUF // 2:3 * NBUF]
        s2sem = bufs_and_sems[3 * NBUF:]
        sid = lax.axis_index("s")
        wid = sid * NC + lax.axis_index("c")
        base = wid * b_per_w

        t_rows = N_TABLE // NS
        pltpu.sync_copy(pe_hbm.at[pl.ds(sid * t_rows, t_rows)],
                        table_sh.at[pl.ds(sid * t_rows, t_rows)])
        pltpu.sync_copy(idx_hbm.at[pl.ds(base, b_per_w)], idx_v)
        plsc.subcore_barrier()

        def gather_desc(b, ci):
            off = pl.multiple_of(ci * CHUNK, CHUNK)
            return pltpu.make_async_copy(
                table_sh.at[idx_v.at[pl.ds(off, CHUNK)]], rows[b], gsem[b])

        def write_desc(b, ci):
            off = pl.multiple_of(ci * CHUNK, CHUNK)
            return pltpu.make_async_copy(
                rows[b], out_hbm.at[pl.ds(base + off, CHUNK)], wsem[b // 2])

        def s1_desc(b):
            return pltpu.make_async_copy(
                rows[b], sbuf_sh.at[sid, _sslot(b)], s1sem[b // 2])

        def s2_desc(b, ci):
            off = pl.multiple_of(ci * CHUNK, CHUNK)
            return pltpu.make_async_copy(
                sbuf_sh.at[sid, _sslot(b)], out_hbm.at[pl.ds(base + off, CHUNK)],
                s2sem[_sslot(b)])

        for b in range(NBUF - LAG):
            gather_desc(b, b).start()

        def group(g, carry):
            for b in range(NBUF):
                i = g * NBUF + b
                b2 = (b + NBUF - LAG) % NBUF
                gather_desc(b, i).wait()
                if b % 2 == 0:
                    write_desc(b, i).start()
                else:
                    @pl.when(i >= 2 * NSB)
                    def _():
                        s2_desc((b + NBUF - 2 * NSB) % NBUF, i - 2 * NSB).wait()

                    s1_desc(b).start()

                @pl.when(i >= LAG)
                def _():
                    if b2 % 2 == 0:
                        write_desc(b2, i - LAG).wait()
                    else:
                        s1_desc(b2).wait()
                        s2_desc(b2, i - LAG).start()

                nxt = i + NBUF - LAG

                @pl.when(nxt < n_chunks)
                def _():
                    gather_desc(b2, nxt).start()
            return carry

        lax.fori_loop(0, n_groups, group, 0)

        for j in range(LAG):
            ci = n_chunks - LAG + j
            b = ci % NBUF
            if b % 2 == 0:
                write_desc(b, ci).wait()
            else:
                s1_desc(b).wait()
                s2_desc(b, ci).start()
        for ci in range(n_chunks - 2 * NSB, n_chunks):
            b = ci % NBUF
            if b % 2 == 1:
                s2_desc(b, ci).wait()

    return gather_kernel


_B_TOTAL = 4096 * 200
_gather = _make_gather(_B_TOTAL)


def kernel(t, pe):
    idx = t.reshape(-1).astype(jnp.int32)
    out = _gather(idx, pe)
    return out.reshape(t.shape + (D_MODEL,))


# speedup vs baseline: 1.5844x; 1.5844x over previous
import functools

import jax
import jax.numpy as jnp
from jax import lax
from jax.experimental import pallas as pl
from jax.experimental.pallas import tpu as pltpu
from jax.experimental.pallas import tpu_sc as plsc

D_MODEL = 128
N_TABLE = 2048
NC, NS = 2, 16
NW = NC * NS
CHUNK = 64
NBUF = 10
LAG = 4


def _make_gather(B):
    b_per_w = B // NW
    n_chunks = b_per_w // CHUNK
    assert n_chunks % NBUF == 0 and n_chunks > NBUF
    n_groups = n_chunks // NBUF
    mesh = plsc.VectorSubcoreMesh(core_axis_name="c", subcore_axis_name="s")

    @functools.partial(
        pl.kernel,
        out_type=jax.ShapeDtypeStruct((B, D_MODEL), jnp.float32),
        mesh=mesh,
        scratch_types=[
            pltpu.VMEM((b_per_w,), jnp.int32),
            pltpu.VMEM_SHARED((N_TABLE, D_MODEL), jnp.float32),
            *[pltpu.VMEM((CHUNK, D_MODEL), jnp.float32) for _ in range(NBUF)],
            *[pltpu.SemaphoreType.DMA for _ in range(2 * NBUF)],
        ],
    )
    def gather_kernel(idx_hbm, pe_hbm, out_hbm, idx_v, table_sh, *bufs_and_sems):
        rows = bufs_and_sems[:NBUF]
        gsem = bufs_and_sems[NBUF:2 * NBUF]
        wsem = bufs_and_sems[2 * NBUF:]
        sid = lax.axis_index("s")
        wid = sid * NC + lax.axis_index("c")
        base = wid * b_per_w

        t_rows = N_TABLE // NS
        pltpu.sync_copy(pe_hbm.at[pl.ds(sid * t_rows, t_rows)],
                        table_sh.at[pl.ds(sid * t_rows, t_rows)])
        pltpu.sync_copy(idx_hbm.at[pl.ds(base, b_per_w)], idx_v)
        plsc.subcore_barrier()

        def gather_desc(b, ci):
            off = pl.multiple_of(ci * CHUNK, CHUNK)
            return pltpu.make_async_copy(
                table_sh.at[idx_v.at[pl.ds(off, CHUNK)]], rows[b], gsem[b])

        def write_desc(b, ci):
            off = pl.multiple_of(ci * CHUNK, CHUNK)
            return pltpu.make_async_copy(
                rows[b], out_hbm.at[pl.ds(base + off, CHUNK)], wsem[b])

        for b in range(NBUF - LAG):
            gather_desc(b, b).start()

        def group(g, carry):
            for b in range(NBUF):
                i = g * NBUF + b
                b2 = (b + NBUF - LAG) % NBUF
                gather_desc(b, i).wait()
                write_desc(b, i).start()

                @pl.when(i >= LAG)
                def _():
                    write_desc(b2, i - LAG).wait()

                nxt = i + NBUF - LAG

                @pl.when(nxt < n_chunks)
                def _():
                    gather_desc(b2, nxt).start()
            return carry

        lax.fori_loop(0, n_groups, group, 0)

        for j in range(LAG):
            ci = n_chunks - LAG + j
            write_desc(ci % NBUF, ci).wait()

    return gather_kernel


_B_TOTAL = 4096 * 200
_gather = _make_gather(_B_TOTAL)


def kernel(t, pe):
    idx = t.reshape(-1).astype(jnp.int32)
    out = _gather(idx, pe)
    return out.reshape(t.shape + (D_MODEL,))
